# baseline (speedup 1.0000x reference)
import functools
import math

import jax
import jax.numpy as jnp
from jax import lax
from jax.experimental import pallas as pl
from jax.experimental.pallas import tpu as pltpu
from jax.experimental.pallas import tpu_sc as plsc

N_HEAD = 12
Q_LEN = 2048
K_LEN = 2048
LINE_LEN = Q_LEN + K_LEN - 1
ROW_PAD = 4232
N_SHIFT = 8


def _line8_body(shift_ref, table_ref, out_ref):
    h = pl.program_id(0)
    r = lax.broadcasted_iota(jnp.int32, (N_SHIFT, ROW_PAD), 0)
    j = lax.broadcasted_iota(jnp.int32, (N_SHIFT, ROW_PAD), 1)
    m = jnp.minimum(r + j, LINE_LEN - 1)
    d_tri = m - (Q_LEN - 1)
    rp = jnp.abs(d_tri + shift_ref[0, 0])
    rp_f = rp.astype(jnp.float32)
    t = jnp.log(rp_f / 8) / math.log(128 / 8) * (16 - 8)
    large = jnp.minimum(8 + t.astype(jnp.int32), 15)
    bucket = jnp.where(rp < 8, rp, large) + jnp.where(d_tri >= 1, 16, 0)
    acc = jnp.zeros((N_SHIFT, ROW_PAD), jnp.float32)
    for b in range(32):
        acc = jnp.where(bucket == b, table_ref[b, h], acc)
    out_ref[0] = acc


def _make_line8(shift, bias_table):
    return pl.pallas_call(
        _line8_body,
        grid=(N_HEAD,),
        in_specs=[
            pl.BlockSpec(memory_space=pltpu.SMEM),
            pl.BlockSpec(memory_space=pltpu.SMEM),
        ],
        out_specs=pl.BlockSpec((1, N_SHIFT, ROW_PAD), lambda h: (h, 0, 0)),
        out_shape=jax.ShapeDtypeStruct((N_HEAD, N_SHIFT, ROW_PAD), jnp.float32),
    )(shift, bias_table)


_N_WORKER = 32
_RPW = N_HEAD * Q_LEN // _N_WORKER
_CHUNK = 16


@functools.cache
def _build_sc_expand():
    mesh = plsc.VectorSubcoreMesh(core_axis_name="c", subcore_axis_name="s")

    @functools.partial(
        pl.kernel,
        mesh=mesh,
        out_type=jax.ShapeDtypeStruct((1, N_HEAD, Q_LEN, K_LEN), jnp.float32),
        compiler_params=pltpu.CompilerParams(use_tc_tiling_on_sc=False),
        scratch_types=[
            pltpu.VMEM((2 * N_SHIFT * ROW_PAD,), jnp.float32),
            pltpu.SemaphoreType.DMA,
        ],
    )
    def _sc_expand(line8_hbm, out_hbm, line_v, sem):
        wid = lax.axis_index("s") * 2 + lax.axis_index("c")
        row0 = wid * _RPW
        h0 = row0 // Q_LEN
        h1 = (row0 + _RPW - 1) // Q_LEN
        line_words = N_SHIFT * ROW_PAD
        pltpu.sync_copy(
            line8_hbm.at[pl.ds(pl.multiple_of(h0 * line_words, 8), line_words)],
            line_v.at[pl.ds(0, line_words)])
        pltpu.sync_copy(
            line8_hbm.at[pl.ds(pl.multiple_of(h1 * line_words, 8), line_words)],
            line_v.at[pl.ds(line_words, line_words)])

        def fire(ci):
            handles = []
            for jj in range(_CHUNK):
                row = row0 + ci * _CHUNK + jj
                h = row // Q_LEN
                st = (Q_LEN - 1) - (row - h * Q_LEN)
                r = lax.rem(st, 8)
                srcoff = pl.multiple_of(
                    (h - h0) * line_words + r * ROW_PAD + (st - r), 8)
                handles.append(
                    pltpu.async_copy(
                        line_v.at[pl.ds(srcoff, K_LEN)],
                        out_hbm.at[0, h, row - h * Q_LEN],
                        sem,
                    )
                )
            return handles

        fire(0)

        def step(i, carry):
            for hd in fire(i + 1):
                hd.wait()
            return carry

        lax.fori_loop(0, _RPW // _CHUNK - 1, step, 0)
        for _ in range(_CHUNK):
            pltpu.make_async_copy(
                line_v.at[pl.ds(0, K_LEN)],
                out_hbm.at[0, 0, 0],
                sem,
            ).wait()

    return _sc_expand


def kernel(q_len, k_len, bias_table):
    shift = jnp.asarray(k_len - q_len, jnp.int32).reshape(1, 1)
    line8 = _make_line8(shift, bias_table)
    return _build_sc_expand()(line8.reshape(N_HEAD * N_SHIFT * ROW_PAD))


# speedup vs baseline: 1.0163x; 1.0163x over previous
import functools
import math

import jax
import jax.numpy as jnp
from jax import lax
from jax.experimental import pallas as pl
from jax.experimental.pallas import tpu as pltpu
from jax.experimental.pallas import tpu_sc as plsc

N_HEAD = 12
Q_LEN = 2048
K_LEN = 2048
LINE_LEN = Q_LEN + K_LEN - 1
ROW_PAD = 4352
N_SHIFT = 8
_CPL = ROW_PAD // 128
_RPH = N_SHIFT * _CPL


def _line8_body(shift_ref, table_ref, out_ref):
    h = pl.program_id(0)
    t = lax.broadcasted_iota(jnp.int32, (_RPH, 128), 0)
    l = lax.broadcasted_iota(jnp.int32, (_RPH, 128), 1)
    r = t // _CPL
    c = t - r * _CPL
    m = jnp.minimum(r + c * 128 + l, LINE_LEN - 1)
    d_tri = m - (Q_LEN - 1)
    rp = jnp.abs(d_tri + shift_ref[0, 0])
    rp_f = rp.astype(jnp.float32)
    t = jnp.log(rp_f / 8) / math.log(128 / 8) * (16 - 8)
    large = jnp.minimum(8 + t.astype(jnp.int32), 15)
    bucket = jnp.where(rp < 8, rp, large) + jnp.where(d_tri >= 1, 16, 0)
    acc = jnp.zeros((_RPH, 128), jnp.float32)
    for b in range(32):
        acc = jnp.where(bucket == b, table_ref[b, h], acc)
    out_ref[...] = acc


def _make_line8(shift, bias_table):
    return pl.pallas_call(
        _line8_body,
        grid=(N_HEAD,),
        in_specs=[
            pl.BlockSpec(memory_space=pltpu.SMEM),
            pl.BlockSpec(memory_space=pltpu.SMEM),
        ],
        out_specs=pl.BlockSpec((_RPH, 128), lambda h: (h, 0)),
        out_shape=jax.ShapeDtypeStruct((N_HEAD * _RPH, 128), jnp.float32),
    )(shift, bias_table)


_N_WORKER = 32
_RPW = N_HEAD * Q_LEN // _N_WORKER
_CHUNK = 16


@functools.cache
def _build_sc_expand():
    mesh = plsc.VectorSubcoreMesh(core_axis_name="c", subcore_axis_name="s")

    @functools.partial(
        pl.kernel,
        mesh=mesh,
        out_type=jax.ShapeDtypeStruct((1, N_HEAD, Q_LEN, K_LEN), jnp.float32),
        compiler_params=pltpu.CompilerParams(use_tc_tiling_on_sc=False),
        scratch_types=[
            pltpu.VMEM((2 * N_SHIFT * ROW_PAD,), jnp.float32),
            pltpu.SemaphoreType.DMA,
        ],
    )
    def _sc_expand(line8_hbm, out_hbm, line_v, sem):
        wid = lax.axis_index("s") * 2 + lax.axis_index("c")
        row0 = wid * _RPW
        h0 = row0 // Q_LEN
        h1 = (row0 + _RPW - 1) // Q_LEN
        line_words = N_SHIFT * ROW_PAD
        pltpu.sync_copy(
            line8_hbm.at[pl.ds(pl.multiple_of(h0 * line_words, 8), line_words)],
            line_v.at[pl.ds(0, line_words)])

        @pl.when(h1 != h0)
        def _stage_second_head():
            pltpu.sync_copy(
                line8_hbm.at[pl.ds(pl.multiple_of(h1 * line_words, 8), line_words)],
                line_v.at[pl.ds(line_words, line_words)])

        def fire(ci):
            handles = []
            for jj in range(_CHUNK):
                row = row0 + ci * _CHUNK + jj
                h = row // Q_LEN
                st = (Q_LEN - 1) - (row - h * Q_LEN)
                r = lax.rem(st, 8)
                srcoff = pl.multiple_of(
                    (h - h0) * line_words + r * ROW_PAD + (st - r), 8)
                handles.append(
                    pltpu.async_copy(
                        line_v.at[pl.ds(srcoff, K_LEN)],
                        out_hbm.at[0, h, row - h * Q_LEN],
                        sem,
                    )
                )
            return handles

        fire(0)

        def step(i, carry):
            for hd in fire(i + 1):
                hd.wait()
            return carry

        lax.fori_loop(0, _RPW // _CHUNK - 1, step, 0)
        for _ in range(_CHUNK):
            pltpu.make_async_copy(
                line_v.at[pl.ds(0, K_LEN)],
                out_hbm.at[0, 0, 0],
                sem,
            ).wait()

    return _sc_expand


def kernel(q_len, k_len, bias_table):
    shift = jnp.asarray(k_len - q_len, jnp.int32).reshape(1, 1)
    line8 = _make_line8(shift, bias_table)
    return _build_sc_expand()(line8.reshape(N_HEAD * N_SHIFT * ROW_PAD))


# speedup vs baseline: 2.1398x; 2.1055x over previous
import functools
import math

import jax
import jax.numpy as jnp
from jax import lax
from jax.experimental import pallas as pl
from jax.experimental.pallas import tpu as pltpu
from jax.experimental.pallas import tpu_sc as plsc

N_HEAD = 12
Q_LEN = 2048
K_LEN = 2048
LINE_LEN = Q_LEN + K_LEN - 1
LINE_PAD = 5120
_QB = Q_LEN // 128


def _bucket_body(shift_ref, out_ref):
    c = lax.broadcasted_iota(jnp.int32, (LINE_PAD // 128, 128), 0)
    l = lax.broadcasted_iota(jnp.int32, (LINE_PAD // 128, 128), 1)
    m = jnp.minimum(c * 128 + l, LINE_LEN - 1)
    d_tri = m - (Q_LEN - 1)
    rp = jnp.abs(d_tri + shift_ref[0, 0])
    rp_f = rp.astype(jnp.float32)
    t = jnp.log(rp_f / 8) / math.log(128 / 8) * (16 - 8)
    large = jnp.minimum(8 + t.astype(jnp.int32), 15)
    out_ref[...] = jnp.where(rp < 8, rp, large) + jnp.where(d_tri >= 1, 16, 0)


def _make_buckets(shift):
    return pl.pallas_call(
        _bucket_body,
        in_specs=[pl.BlockSpec(memory_space=pltpu.SMEM)],
        out_specs=pl.BlockSpec((LINE_PAD // 128, 128), lambda: (0, 0)),
        out_shape=jax.ShapeDtypeStruct((LINE_PAD // 128, 128), jnp.int32),
    )(shift)


_HALF_J = LINE_PAD // 2


@functools.cache
def _build_sc_lookup():
    mesh = plsc.VectorSubcoreMesh(core_axis_name="c", subcore_axis_name="s")

    @functools.partial(
        pl.kernel,
        mesh=mesh,
        out_type=jax.ShapeDtypeStruct((N_HEAD * LINE_PAD,), jnp.float32),
        compiler_params=pltpu.CompilerParams(
            use_tc_tiling_on_sc=False, needs_layout_passes=False),
        scratch_types=[
            pltpu.VMEM((32 * N_HEAD,), jnp.float32),
            pltpu.VMEM((LINE_PAD,), jnp.int32),
            pltpu.VMEM((LINE_PAD,), jnp.float32),
        ],
    )
    def _sc_lookup(buckets_hbm, table_hbm, out_hbm, table_v, buckets_v, line_v):
        wid = lax.axis_index("s") * 2 + lax.axis_index("c")

        @pl.when(wid < N_HEAD * 2)
        def _():
            h = wid // 2
            j0 = (wid % 2) * _HALF_J
            pltpu.sync_copy(table_hbm, table_v)
            pltpu.sync_copy(buckets_hbm, buckets_v)

            def step(i, carry):
                j = j0 + i * 16
                b = buckets_v[pl.ds(j, 16)]
                vals = plsc.load_gather(table_v, [b * N_HEAD + h])
                line_v[pl.ds(j, 16)] = vals
                return carry

            lax.fori_loop(0, _HALF_J // 16, step, 0)
            pltpu.sync_copy(
                line_v.at[pl.ds(pl.multiple_of(j0, 8), _HALF_J)],
                out_hbm.at[pl.ds(pl.multiple_of(h * LINE_PAD + j0, 8), _HALF_J)],
            )

    return _sc_lookup


def _expand_body(line_ref, out_ref, shifted_ref):
    qb = pl.program_id(1)

    @pl.when(qb == 0)
    def _build_bank():
        for s in range(128):
            shifted_ref[s, :] = line_ref[pl.ds(127 - s, 4096)]

    off = pl.multiple_of(128 * (_QB - 1 - qb), 128)
    out_ref[0, 0] = shifted_ref[:, pl.ds(off, K_LEN)]


def _make_expand(line_flat):
    return pl.pallas_call(
        _expand_body,
        grid=(N_HEAD, _QB),
        in_specs=[pl.BlockSpec((LINE_PAD,), lambda h, qb: (h,))],
        out_specs=pl.BlockSpec((1, 1, 128, K_LEN), lambda h, qb: (0, h, qb, 0)),
        out_shape=jax.ShapeDtypeStruct((1, N_HEAD, Q_LEN, K_LEN), jnp.float32),
        scratch_shapes=[pltpu.VMEM((128, 4096), jnp.float32)],
    )(line_flat)


def kernel(q_len, k_len, bias_table):
    shift = jnp.asarray(k_len - q_len, jnp.int32).reshape(1, 1)
    buckets = _make_buckets(shift)
    line_flat = _build_sc_lookup()(
        buckets.reshape(LINE_PAD), bias_table.reshape(32 * N_HEAD))
    return _make_expand(line_flat)


# speedup vs baseline: 3.4127x; 1.5948x over previous
import functools
import math

import jax
import jax.numpy as jnp
from jax import lax
from jax.experimental import pallas as pl
from jax.experimental.pallas import tpu as pltpu
from jax.experimental.pallas import tpu_sc as plsc

N_HEAD = 12
Q_LEN = 2048
K_LEN = 2048
LINE_LEN = Q_LEN + K_LEN - 1
LINE_PAD = 5120
_QB = Q_LEN // 128


def _bucket_body(shift_ref, out_ref):
    c = lax.broadcasted_iota(jnp.int32, (LINE_PAD // 128, 128), 0)
    l = lax.broadcasted_iota(jnp.int32, (LINE_PAD // 128, 128), 1)
    m = jnp.minimum(c * 128 + l, LINE_LEN - 1)
    d_tri = m - (Q_LEN - 1)
    rp = jnp.abs(d_tri + shift_ref[0, 0])
    rp_f = rp.astype(jnp.float32)
    t = jnp.log(rp_f / 8) / math.log(128 / 8) * (16 - 8)
    large = jnp.minimum(8 + t.astype(jnp.int32), 15)
    out_ref[...] = jnp.where(rp < 8, rp, large) + jnp.where(d_tri >= 1, 16, 0)


def _make_buckets(shift):
    return pl.pallas_call(
        _bucket_body,
        in_specs=[pl.BlockSpec(memory_space=pltpu.SMEM)],
        out_specs=pl.BlockSpec((LINE_PAD // 128, 128), lambda: (0, 0)),
        out_shape=jax.ShapeDtypeStruct((LINE_PAD // 128, 128), jnp.int32),
    )(shift)


_HALF_J = LINE_PAD // 2


@functools.cache
def _build_sc_lookup():
    mesh = plsc.VectorSubcoreMesh(core_axis_name="c", subcore_axis_name="s")

    @functools.partial(
        pl.kernel,
        mesh=mesh,
        out_type=jax.ShapeDtypeStruct((N_HEAD * LINE_PAD,), jnp.float32),
        compiler_params=pltpu.CompilerParams(
            use_tc_tiling_on_sc=False, needs_layout_passes=False),
        scratch_types=[
            pltpu.VMEM((32 * N_HEAD,), jnp.float32),
            pltpu.VMEM((LINE_PAD,), jnp.int32),
            pltpu.VMEM((LINE_PAD,), jnp.float32),
        ],
    )
    def _sc_lookup(buckets_hbm, table_hbm, out_hbm, table_v, buckets_v, line_v):
        wid = lax.axis_index("s") * 2 + lax.axis_index("c")

        @pl.when(wid < N_HEAD * 2)
        def _():
            h = wid // 2
            j0 = (wid % 2) * _HALF_J
            pltpu.sync_copy(table_hbm, table_v)
            pltpu.sync_copy(buckets_hbm, buckets_v)

            def step(i, carry):
                j = j0 + i * 16
                b = buckets_v[pl.ds(j, 16)]
                vals = plsc.load_gather(table_v, [b * N_HEAD + h])
                line_v[pl.ds(j, 16)] = vals
                return carry

            lax.fori_loop(0, _HALF_J // 16, step, 0)
            pltpu.sync_copy(
                line_v.at[pl.ds(pl.multiple_of(j0, 8), _HALF_J)],
                out_hbm.at[pl.ds(pl.multiple_of(h * LINE_PAD + j0, 8), _HALF_J)],
            )

    return _sc_lookup


def _expand_body(line_ref, out_ref, bank_ref, sem):
    h = pl.program_id(0)
    p = lax.rem(h, 2)

    def _copies(head, bank_slot):
        cps = []
        for qb in range(_QB):
            src = bank_ref.at[bank_slot, :, pl.ds(128 * (_QB - 1 - qb), K_LEN)]
            dst = out_ref.at[0, head, pl.ds(128 * qb, 128), :]
            cps.append(pltpu.make_async_copy(src, dst, sem))
        return cps

    @pl.when(h >= 2)
    def _drain_two_back():
        for cp in _copies(h - 2, p):
            cp.wait()

    for s in range(128):
        bank_ref[p, s, :] = line_ref[pl.ds(127 - s, 4096)]

    for cp in _copies(h, p):
        cp.start()

    @pl.when(h == N_HEAD - 1)
    def _drain_tail():
        for cp in _copies(h - 1, 1 - p) + _copies(h, p):
            cp.wait()


def _make_expand(line_flat):
    return pl.pallas_call(
        _expand_body,
        grid=(N_HEAD,),
        in_specs=[pl.BlockSpec((LINE_PAD,), lambda h: (h,))],
        out_specs=pl.BlockSpec(memory_space=pl.ANY),
        out_shape=jax.ShapeDtypeStruct((1, N_HEAD, Q_LEN, K_LEN), jnp.float32),
        scratch_shapes=[
            pltpu.VMEM((2, 128, 4096), jnp.float32),
            pltpu.SemaphoreType.DMA,
        ],
    )(line_flat)


def kernel(q_len, k_len, bias_table):
    shift = jnp.asarray(k_len - q_len, jnp.int32).reshape(1, 1)
    buckets = _make_buckets(shift)
    line_flat = _build_sc_lookup()(
        buckets.reshape(LINE_PAD), bias_table.reshape(32 * N_HEAD))
    return _make_expand(line_flat)


# speedup vs baseline: 3.4182x; 1.0016x over previous
import functools
import math

import jax
import jax.numpy as jnp
from jax import lax
from jax.experimental import pallas as pl
from jax.experimental.pallas import tpu as pltpu
from jax.experimental.pallas import tpu_sc as plsc

N_HEAD = 12
Q_LEN = 2048
K_LEN = 2048
LINE_LEN = Q_LEN + K_LEN - 1
LINE_PAD = 5120
_QB = Q_LEN // 128


def _bucket_body(shift_ref, out_ref):
    c = lax.broadcasted_iota(jnp.int32, (LINE_PAD // 128, 128), 0)
    l = lax.broadcasted_iota(jnp.int32, (LINE_PAD // 128, 128), 1)
    m = jnp.minimum(c * 128 + l, LINE_LEN - 1)
    d_tri = m - (Q_LEN - 1)
    rp = jnp.abs(d_tri + shift_ref[0, 0])
    rp_f = rp.astype(jnp.float32)
    t = jnp.log(rp_f / 8) / math.log(128 / 8) * (16 - 8)
    large = jnp.minimum(8 + t.astype(jnp.int32), 15)
    out_ref[...] = jnp.where(rp < 8, rp, large) + jnp.where(d_tri >= 1, 16, 0)


def _make_buckets(shift):
    return pl.pallas_call(
        _bucket_body,
        in_specs=[pl.BlockSpec(memory_space=pltpu.SMEM)],
        out_specs=pl.BlockSpec((LINE_PAD // 128, 128), lambda: (0, 0)),
        out_shape=jax.ShapeDtypeStruct((LINE_PAD // 128, 128), jnp.int32),
    )(shift)


_HALF_J = LINE_PAD // 2


@functools.cache
def _build_sc_lookup():
    mesh = plsc.VectorSubcoreMesh(core_axis_name="c", subcore_axis_name="s")

    @functools.partial(
        pl.kernel,
        mesh=mesh,
        out_type=jax.ShapeDtypeStruct((N_HEAD * LINE_PAD,), jnp.float32),
        compiler_params=pltpu.CompilerParams(
            use_tc_tiling_on_sc=False, needs_layout_passes=False),
        scratch_types=[
            pltpu.VMEM((32 * N_HEAD,), jnp.float32),
            pltpu.VMEM((LINE_PAD,), jnp.int32),
            pltpu.VMEM((LINE_PAD,), jnp.float32),
        ],
    )
    def _sc_lookup(buckets_hbm, table_hbm, out_hbm, table_v, buckets_v, line_v):
        wid = lax.axis_index("s") * 2 + lax.axis_index("c")

        @pl.when(wid < N_HEAD * 2)
        def _():
            h = wid // 2
            j0 = (wid % 2) * _HALF_J
            pltpu.sync_copy(table_hbm, table_v)
            pltpu.sync_copy(buckets_hbm, buckets_v)

            def step(i, carry):
                j = j0 + i * 16
                b = buckets_v[pl.ds(j, 16)]
                vals = plsc.load_gather(table_v, [b * N_HEAD + h])
                line_v[pl.ds(j, 16)] = vals
                return carry

            lax.fori_loop(0, _HALF_J // 16, step, 0)
            pltpu.sync_copy(
                line_v.at[pl.ds(pl.multiple_of(j0, 8), _HALF_J)],
                out_hbm.at[pl.ds(pl.multiple_of(h * LINE_PAD + j0, 8), _HALF_J)],
            )

    return _sc_lookup


def _expand_body(line_ref, out_ref, bank_ref, sem):
    h = pl.program_id(0)
    p = lax.rem(h, 2)

    def _copies(head, bank_slot):
        cps = []
        for qb in range(_QB):
            src = bank_ref.at[bank_slot, :, pl.ds(128 * (_QB - 1 - qb), K_LEN)]
            dst = out_ref.at[0, head, pl.ds(128 * qb, 128), :]
            cps.append(pltpu.make_async_copy(src, dst, sem.at[bank_slot]))
        return cps

    @pl.when(h >= 2)
    def _drain_two_back():
        for cp in _copies(h - 2, p):
            cp.wait()

    for s in range(128):
        bank_ref[p, s, :] = line_ref[pl.ds(127 - s, 4096)]

    for cp in _copies(h, p):
        cp.start()

    @pl.when(h == N_HEAD - 1)
    def _drain_tail():
        for cp in _copies(h - 1, 1 - p) + _copies(h, p):
            cp.wait()


def _make_expand(line_flat):
    return pl.pallas_call(
        _expand_body,
        grid=(N_HEAD,),
        in_specs=[pl.BlockSpec((LINE_PAD,), lambda h: (h,))],
        out_specs=pl.BlockSpec(memory_space=pl.ANY),
        out_shape=jax.ShapeDtypeStruct((1, N_HEAD, Q_LEN, K_LEN), jnp.float32),
        scratch_shapes=[
            pltpu.VMEM((2, 128, 4096), jnp.float32),
            pltpu.SemaphoreType.DMA((2,)),
        ],
    )(line_flat)


def kernel(q_len, k_len, bias_table):
    shift = jnp.asarray(k_len - q_len, jnp.int32).reshape(1, 1)
    buckets = _make_buckets(shift)
    line_flat = _build_sc_lookup()(
        buckets.reshape(LINE_PAD), bias_table.reshape(32 * N_HEAD))
    return _make_expand(line_flat)
